# baseline (speedup 1.0000x reference)
import functools

import jax
import jax.numpy as jnp
from jax import lax
from jax.experimental import pallas as pl
from jax.experimental.pallas import tpu as pltpu
from jax.experimental.pallas import tpu_sc as plsc

NUM_TIMESTEPS = 1000
BATCH_SIZE = 4096

_info = plsc.get_sparse_core_info()
_NC, _NS, _L = _info.num_cores, _info.num_subcores, _info.num_lanes
_NW = _NC * _NS
_BPW = BATCH_SIZE // _NW

_mesh = plsc.VectorSubcoreMesh(core_axis_name="c", subcore_axis_name="s")


@functools.partial(
    pl.kernel,
    mesh=_mesh,
    out_type=(
        jax.ShapeDtypeStruct((BATCH_SIZE,), jnp.float32),
        jax.ShapeDtypeStruct((BATCH_SIZE,), jnp.float32),
    ),
    scratch_types=[
        pltpu.VMEM((_BPW,), jnp.int32),
        pltpu.VMEM((_BPW,), jnp.float32),
        pltpu.VMEM((_BPW,), jnp.float32),
        pltpu.SemaphoreType.DMA,
        pltpu.SemaphoreType.DMA,
    ],
)
def _ddpm_gather(t_hbm, beta_hbm, alpha_hbm, beta_out, alpha_out,
                 t_v, bout_v, aout_v, sem_b, sem_a):
    wid = lax.axis_index("s") * _NC + lax.axis_index("c")
    base = wid * _BPW
    pltpu.sync_copy(t_hbm.at[pl.ds(base, _BPW)], t_v)
    cp_b = pltpu.async_copy(beta_hbm.at[t_v], bout_v, sem_b)
    cp_a = pltpu.async_copy(alpha_hbm.at[t_v], aout_v, sem_a)
    cp_b.wait()
    st_b = pltpu.async_copy(bout_v, beta_out.at[pl.ds(base, _BPW)], sem_b)
    cp_a.wait()
    st_a = pltpu.async_copy(aout_v, alpha_out.at[pl.ds(base, _BPW)], sem_a)
    st_b.wait()
    st_a.wait()


def kernel(t, beta, alpha):
    return _ddpm_gather(t.astype(jnp.int32), beta, alpha)


# speedup vs baseline: 1.0447x; 1.0447x over previous
import functools

import jax
import jax.numpy as jnp
from jax import lax
from jax.experimental import pallas as pl
from jax.experimental.pallas import tpu as pltpu
from jax.experimental.pallas import tpu_sc as plsc

NUM_TIMESTEPS = 1000
BATCH_SIZE = 4096

_info = plsc.get_sparse_core_info()
_NC, _NS, _L = _info.num_cores, _info.num_subcores, _info.num_lanes
_NW = _NS
_BPW = BATCH_SIZE // _NW

_mesh = plsc.VectorSubcoreMesh(
    core_axis_name="c", subcore_axis_name="s", num_cores=1)


@functools.partial(
    pl.kernel,
    mesh=_mesh,
    out_type=(
        jax.ShapeDtypeStruct((BATCH_SIZE,), jnp.float32),
        jax.ShapeDtypeStruct((BATCH_SIZE,), jnp.float32),
    ),
    scratch_types=[
        pltpu.VMEM((_BPW,), jnp.int32),
        pltpu.VMEM((_BPW,), jnp.float32),
        pltpu.VMEM((_BPW,), jnp.float32),
        pltpu.SemaphoreType.DMA,
        pltpu.SemaphoreType.DMA,
    ],
)
def _ddpm_gather(t_hbm, beta_hbm, alpha_hbm, beta_out, alpha_out,
                 t_v, bout_v, aout_v, sem_b, sem_a):
    wid = lax.axis_index("s")
    base = wid * _BPW
    pltpu.sync_copy(t_hbm.at[pl.ds(base, _BPW)], t_v)
    cp_b = pltpu.async_copy(beta_hbm.at[t_v], bout_v, sem_b)
    cp_a = pltpu.async_copy(alpha_hbm.at[t_v], aout_v, sem_a)
    cp_b.wait()
    st_b = pltpu.async_copy(bout_v, beta_out.at[pl.ds(base, _BPW)], sem_b)
    cp_a.wait()
    st_a = pltpu.async_copy(aout_v, alpha_out.at[pl.ds(base, _BPW)], sem_a)
    st_b.wait()
    st_a.wait()


def kernel(t, beta, alpha):
    return _ddpm_gather(t.astype(jnp.int32), beta, alpha)
